# baseline (speedup 1.0000x reference)
import jax
import jax.numpy as jnp
from jax.experimental import pallas as pl
from jax.experimental.pallas import tpu as pltpu


def _pick_tile(n, preferred, multiple):
    if n <= preferred:
        return n
    t = (preferred // multiple) * multiple
    while t >= multiple:
        if n % t == 0:
            return t
        t -= multiple
    return n


def _pooled_head_1pass_kernel(x_ref, m_ref, w_ref, *out_refs):
    x = x_ref[...].astype(jnp.float32)
    v = jnp.where(m_ref[...], 0.0, 1.0)
    acc = jnp.sum(x * v[:, :, None], axis=1)
    cnt = jnp.sum(v, axis=1, keepdims=True)
    inv = 1.0 / jnp.maximum(cnt, 1.0)
    out = jax.lax.dot_general(
        acc * inv, w_ref[...],
        dimension_numbers=(((1,), (1,)), ((), ())),
        preferred_element_type=jnp.float32,
    )
    for i, oref in enumerate(out_refs):
        oref[...] = out[:, i:i + 1].astype(oref.dtype)


def _pooled_head_acc_kernel(x_ref, m_ref, w_ref, *refs):
    out_refs = refs[:-2]
    acc_ref, cnt_ref = refs[-2:]
    s = pl.program_id(1)

    @pl.when(s == 0)
    def _():
        acc_ref[...] = jnp.zeros_like(acc_ref)
        cnt_ref[...] = jnp.zeros_like(cnt_ref)

    x = x_ref[...].astype(jnp.float32)
    v = jnp.where(m_ref[...], 0.0, 1.0)
    acc_ref[...] += jnp.sum(x * v[:, :, None], axis=1)
    cnt_ref[...] += jnp.sum(v, axis=1, keepdims=True)

    @pl.when(s == pl.num_programs(1) - 1)
    def _():
        inv = 1.0 / jnp.maximum(cnt_ref[...], 1.0)
        out = jax.lax.dot_general(
            acc_ref[...] * inv, w_ref[...],
            dimension_numbers=(((1,), (1,)), ((), ())),
            preferred_element_type=jnp.float32,
        )
        for i, oref in enumerate(out_refs):
            oref[...] = out[:, i:i + 1].astype(oref.dtype)


def kernel(x, weight, mask):
    B, S, D = x.shape
    T = weight.shape[0]
    out_dtype = jnp.promote_types(x.dtype, weight.dtype)
    itemsize = jnp.dtype(x.dtype).itemsize

    out_shape = [jax.ShapeDtypeStruct((B, 1), out_dtype) for _ in range(T)]
    w_spec = pl.BlockSpec((T, D), lambda b, *_: (0,) * (1 + len(_)))

    row_bytes = S * D * itemsize
    B_1pass = _pick_tile(B, max(8, (16 * 1024 * 1024) // max(1, row_bytes)), 8)

    if B_1pass * row_bytes <= 24 * 1024 * 1024:
        B_tile = B_1pass
        out = pl.pallas_call(
            _pooled_head_1pass_kernel,
            out_shape=out_shape,
            grid=(B // B_tile,),
            in_specs=[
                pl.BlockSpec((B_tile, S, D), lambda b: (b, 0, 0)),
                pl.BlockSpec((B_tile, S), lambda b: (b, 0)),
                pl.BlockSpec((T, D), lambda b: (0, 0)),
            ],
            out_specs=[pl.BlockSpec((B_tile, 1), lambda b: (b, 0)) for _ in range(T)],
            compiler_params=pltpu.CompilerParams(
                dimension_semantics=("parallel",),
                vmem_limit_bytes=int(min(2 * B_tile * row_bytes + (8 << 20), 100 << 20)),
            ),
        )(x, mask, weight)
    else:
        B_tile = _pick_tile(B, 32, 32 if B % 32 == 0 else 8)
        s_budget = max(128, (16 * 1024 * 1024) // max(1, B_tile * D * itemsize))
        S_tile = _pick_tile(S, s_budget, 128 if S % 128 == 0 else 8)
        out = pl.pallas_call(
            _pooled_head_acc_kernel,
            out_shape=out_shape,
            grid=(B // B_tile, S // S_tile),
            in_specs=[
                pl.BlockSpec((B_tile, S_tile, D), lambda b, s: (b, s, 0)),
                pl.BlockSpec((B_tile, S_tile), lambda b, s: (b, s)),
                pl.BlockSpec((T, D), lambda b, s: (0, 0)),
            ],
            out_specs=[pl.BlockSpec((B_tile, 1), lambda b, s: (b, 0)) for _ in range(T)],
            scratch_shapes=[
                pltpu.VMEM((B_tile, D), jnp.float32),
                pltpu.VMEM((B_tile, 1), jnp.float32),
            ],
            compiler_params=pltpu.CompilerParams(
                dimension_semantics=("parallel", "arbitrary"),
                vmem_limit_bytes=int(
                    min(2 * B_tile * S_tile * D * itemsize + (8 << 20), 100 << 20)
                ),
            ),
        )(x, mask, weight)

    return {f"t{i}": out[i] for i in range(T)}


# speedup vs baseline: 1.0065x; 1.0065x over previous
import jax
import jax.numpy as jnp
from jax.experimental import pallas as pl
from jax.experimental.pallas import tpu as pltpu


def _pick_tile(n, preferred, multiple):
    if n <= preferred:
        return n
    t = (preferred // multiple) * multiple
    while t >= multiple:
        if n % t == 0:
            return t
        t -= multiple
    return n


def _pooled_head_kernel(x_ref, m_ref, w_ref, *refs):
    out_refs = refs[:-2]
    acc_ref, cnt_ref = refs[-2:]
    s = pl.program_id(1)

    @pl.when(s == 0)
    def _():
        acc_ref[...] = jnp.zeros_like(acc_ref)
        cnt_ref[...] = jnp.zeros_like(cnt_ref)

    x = x_ref[...].astype(jnp.float32)
    v = jnp.where(m_ref[...], 0.0, 1.0)
    acc_ref[...] += jnp.sum(x * v[:, :, None], axis=1)
    cnt_ref[...] += jnp.sum(v, axis=1, keepdims=True)

    @pl.when(s == pl.num_programs(1) - 1)
    def _():
        inv = 1.0 / jnp.maximum(cnt_ref[...], 1.0)
        pooled = acc_ref[...] * inv
        out = jax.lax.dot_general(
            pooled, w_ref[...],
            dimension_numbers=(((1,), (1,)), ((), ())),
            preferred_element_type=jnp.float32,
        )
        for i, oref in enumerate(out_refs):
            oref[...] = out[:, i:i + 1].astype(oref.dtype)


def kernel(x, weight, mask):
    B, S, D = x.shape
    T = weight.shape[0]
    out_dtype = jnp.promote_types(x.dtype, weight.dtype)
    itemsize = jnp.dtype(x.dtype).itemsize

    B_tile = _pick_tile(B, 32, 32 if B % 32 == 0 else 8)
    s_budget = max(128, (16 * 1024 * 1024) // max(1, B_tile * D * itemsize))
    S_tile = _pick_tile(S, s_budget, 128 if S % 128 == 0 else 8)
    grid = (B // B_tile, S // S_tile)

    x_bytes = B_tile * S_tile * D * itemsize
    vmem_limit = int(min(2 * x_bytes + (8 << 20), 100 << 20))

    out = pl.pallas_call(
        _pooled_head_kernel,
        out_shape=[jax.ShapeDtypeStruct((B, 1), out_dtype) for _ in range(T)],
        grid=grid,
        in_specs=[
            pl.BlockSpec((B_tile, S_tile, D), lambda b, s: (b, s, 0)),
            pl.BlockSpec((B_tile, S_tile), lambda b, s: (b, s)),
            pl.BlockSpec((T, D), lambda b, s: (0, 0)),
        ],
        out_specs=[pl.BlockSpec((B_tile, 1), lambda b, s: (b, 0)) for _ in range(T)],
        scratch_shapes=[
            pltpu.VMEM((B_tile, D), jnp.float32),
            pltpu.VMEM((B_tile, 1), jnp.float32),
        ],
        compiler_params=pltpu.CompilerParams(
            dimension_semantics=("parallel", "arbitrary"),
            vmem_limit_bytes=vmem_limit,
        ),
    )(x, mask, weight)

    return {f"t{i}": out[i] for i in range(T)}
